# Initial kernel scaffold; baseline (speedup 1.0000x reference)
import jax
import jax.numpy as jnp
from jax.experimental import pallas as pl


def kernel(indices, table):
    raise NotImplementedError("write your pallas kernel here")


# speedup vs baseline: 2.9677x; 2.9677x over previous
import functools

import jax
import jax.numpy as jnp
from jax import lax
from jax.experimental import pallas as pl
from jax.experimental.pallas import tpu as pltpu
from jax.experimental.pallas import tpu_sc as plsc

D = 128
NC = 2
NS = 16
NW = NC * NS
CHUNK = 128


def _make_gather(b_flat: int):
    rows_per_w = b_flat // NW
    nchunk = rows_per_w // CHUNK
    mesh = plsc.VectorSubcoreMesh(core_axis_name="c", subcore_axis_name="s")

    @functools.partial(
        pl.kernel,
        mesh=mesh,
        out_type=jax.ShapeDtypeStruct((b_flat, D), jnp.float32),
        scratch_types=[
            pltpu.VMEM((nchunk, CHUNK), jnp.int32),
            pltpu.VMEM((CHUNK, D), jnp.float32),
            pltpu.VMEM((CHUNK, D), jnp.float32),
            pltpu.SemaphoreType.DMA,
            pltpu.SemaphoreType.DMA,
        ],
    )
    def gather_kernel(table_hbm, idx_hbm, out_hbm, idx_v, buf0, buf1, sem0, sem1):
        wid = lax.axis_index("s") * NC + lax.axis_index("c")
        base = wid * rows_per_w
        pltpu.sync_copy(idx_hbm.at[wid], idx_v)

        def body(j, _):
            copy = pltpu.async_copy(table_hbm.at[idx_v.at[j]], buf0, sem0)
            copy.wait()
            pltpu.sync_copy(buf0, out_hbm.at[pl.ds(base + j * CHUNK, CHUNK)])
            return _

        lax.fori_loop(0, nchunk, body, 0, unroll=False)

    return gather_kernel


def kernel(indices, table):
    b, h = indices.shape
    b_flat = b * h
    idx2d = indices.reshape(NW, b_flat // (NW * CHUNK), CHUNK)
    out = _make_gather(b_flat)(table, idx2d)
    return out.reshape(b, h, D)


# speedup vs baseline: 3.3448x; 1.1271x over previous
import functools

import jax
import jax.numpy as jnp
from jax import lax
from jax.experimental import pallas as pl
from jax.experimental.pallas import tpu as pltpu
from jax.experimental.pallas import tpu_sc as plsc

D = 128
NC = 2
NS = 16
NW = NC * NS
CHUNK = 128


def _make_gather(b_flat: int):
    rows_per_w = b_flat // NW
    nchunk = rows_per_w // CHUNK
    mesh = plsc.VectorSubcoreMesh(core_axis_name="c", subcore_axis_name="s")

    nbuf = 5
    assert nchunk % nbuf == 0
    nround = nchunk // nbuf

    @functools.partial(
        pl.kernel,
        mesh=mesh,
        out_type=jax.ShapeDtypeStruct((b_flat, D), jnp.float32),
        scratch_types=[
            pltpu.VMEM((nchunk, CHUNK), jnp.int32),
        ]
        + [pltpu.VMEM((CHUNK, D), jnp.float32) for _ in range(nbuf)]
        + [pltpu.SemaphoreType.DMA for _ in range(2 * nbuf)],
    )
    def gather_kernel(table_hbm, idx_hbm, out_hbm, idx_v, *scratch):
        bufs = scratch[:nbuf]
        gsems = scratch[nbuf : 2 * nbuf]
        ssems = scratch[2 * nbuf :]
        wid = lax.axis_index("s") * NC + lax.axis_index("c")
        base = wid * rows_per_w
        pltpu.sync_copy(idx_hbm.at[wid], idx_v)

        for b in range(nbuf):
            pltpu.async_copy(table_hbm.at[idx_v.at[b]], bufs[b], gsems[b])

        def round_body(g, _):
            for b in range(nbuf):
                j = g * nbuf + b
                pltpu.make_async_copy(table_hbm.at[idx_v.at[j]], bufs[b], gsems[b]).wait()
                st = pltpu.async_copy(
                    bufs[b], out_hbm.at[pl.ds(base + j * CHUNK, CHUNK)], ssems[b]
                )
                st.wait()
                pltpu.async_copy(table_hbm.at[idx_v.at[j + nbuf]], bufs[b], gsems[b])
            return _

        lax.fori_loop(0, nround - 1, round_body, 0, unroll=False)

        for b in range(nbuf):
            j = (nround - 1) * nbuf + b
            pltpu.make_async_copy(table_hbm.at[idx_v.at[j]], bufs[b], gsems[b]).wait()
            pltpu.sync_copy(bufs[b], out_hbm.at[pl.ds(base + j * CHUNK, CHUNK)])

    return gather_kernel


def kernel(indices, table):
    b, h = indices.shape
    b_flat = b * h
    idx2d = indices.reshape(NW, b_flat // (NW * CHUNK), CHUNK)
    out = _make_gather(b_flat)(table, idx2d)
    return out.reshape(b, h, D)


# speedup vs baseline: 5.9596x; 1.7818x over previous
import functools

import jax
import jax.numpy as jnp
from jax import lax
from jax.experimental import pallas as pl
from jax.experimental.pallas import tpu as pltpu
from jax.experimental.pallas import tpu_sc as plsc

NC = 2
NS = 16
NW = NC * NS


def _make_gather(batch: int, hist: int, vocab: int, d: int):
    rows_per_w = batch // NW
    nbuf = 8
    assert rows_per_w % nbuf == 0
    nround = rows_per_w // nbuf
    mesh = plsc.VectorSubcoreMesh(core_axis_name="c", subcore_axis_name="s")

    @functools.partial(
        pl.kernel,
        mesh=mesh,
        out_type=jax.ShapeDtypeStruct((batch, hist, d), jnp.float32),
        scratch_types=[
            pltpu.VMEM((rows_per_w, hist), jnp.int32),
        ]
        + [pltpu.VMEM((hist, d), jnp.float32) for _ in range(nbuf)]
        + [pltpu.SemaphoreType.DMA for _ in range(2 * nbuf)],
    )
    def gather_kernel(table_hbm, idx_hbm, out_hbm, idx_v, *scratch):
        bufs = scratch[:nbuf]
        gsems = scratch[nbuf : 2 * nbuf]
        ssems = scratch[2 * nbuf :]
        wid = lax.axis_index("s") * NC + lax.axis_index("c")
        base = wid * rows_per_w
        pltpu.sync_copy(idx_hbm.at[pl.ds(base, rows_per_w)], idx_v)

        for b in range(nbuf):
            pltpu.async_copy(table_hbm.at[idx_v.at[b]], bufs[b], gsems[b])

        def round_body(g, _):
            for b in range(nbuf):
                i = g * nbuf + b
                pltpu.make_async_copy(table_hbm.at[idx_v.at[i]], bufs[b], gsems[b]).wait()
                st = pltpu.async_copy(bufs[b], out_hbm.at[base + i], ssems[b])
                st.wait()
                pltpu.async_copy(table_hbm.at[idx_v.at[i + nbuf]], bufs[b], gsems[b])
            return _

        lax.fori_loop(0, nround - 1, round_body, 0, unroll=False)

        for b in range(nbuf):
            i = (nround - 1) * nbuf + b
            pltpu.make_async_copy(table_hbm.at[idx_v.at[i]], bufs[b], gsems[b]).wait()
            pltpu.sync_copy(bufs[b], out_hbm.at[base + i])

    return gather_kernel


def kernel(indices, table):
    b, h = indices.shape
    v, d = table.shape
    return _make_gather(b, h, v, d)(table, indices)


# speedup vs baseline: 5.9998x; 1.0067x over previous
import functools

import jax
import jax.numpy as jnp
from jax import lax
from jax.experimental import pallas as pl
from jax.experimental.pallas import tpu as pltpu
from jax.experimental.pallas import tpu_sc as plsc

NC = 2
NS = 16
NW = NC * NS


def _make_gather(batch: int, hist: int, vocab: int, d: int):
    rows_per_w = batch // NW
    nbuf = 8
    pair = 2
    npair = rows_per_w // pair
    assert npair % nbuf == 0
    nround = npair // nbuf
    mesh = plsc.VectorSubcoreMesh(core_axis_name="c", subcore_axis_name="s")

    @functools.partial(
        pl.kernel,
        mesh=mesh,
        out_type=jax.ShapeDtypeStruct((batch, hist, d), jnp.float32),
        scratch_types=[
            pltpu.VMEM((rows_per_w, hist), jnp.int32),
        ]
        + [pltpu.VMEM((pair, hist, d), jnp.float32) for _ in range(nbuf)]
        + [pltpu.SemaphoreType.DMA for _ in range(2 * nbuf)],
    )
    def gather_kernel(table_hbm, idx_hbm, out_hbm, idx_v, *scratch):
        bufs = scratch[:nbuf]
        gsems = scratch[nbuf : 2 * nbuf]
        ssems = scratch[2 * nbuf :]
        wid = lax.axis_index("s") * NC + lax.axis_index("c")
        base = wid * rows_per_w
        pltpu.sync_copy(idx_hbm.at[pl.ds(base, rows_per_w)], idx_v)

        def start_pair(p, b):
            for k in range(pair):
                pltpu.async_copy(
                    table_hbm.at[idx_v.at[p * pair + k]], bufs[b].at[k], gsems[b]
                )

        def wait_pair(p, b):
            for k in range(pair):
                pltpu.make_async_copy(
                    table_hbm.at[idx_v.at[p * pair + k]], bufs[b].at[k], gsems[b]
                ).wait()

        for b in range(nbuf):
            start_pair(b, b)

        def round_body(g, _):
            for b in range(nbuf):
                p = g * nbuf + b
                wait_pair(p, b)
                st = pltpu.async_copy(
                    bufs[b], out_hbm.at[pl.ds(base + p * pair, pair)], ssems[b]
                )
                st.wait()
                start_pair(p + nbuf, b)
            return _

        lax.fori_loop(0, nround - 1, round_body, 0, unroll=False)

        for b in range(nbuf):
            p = (nround - 1) * nbuf + b
            wait_pair(p, b)
            pltpu.sync_copy(bufs[b], out_hbm.at[pl.ds(base + p * pair, pair)])

    return gather_kernel


def kernel(indices, table):
    b, h = indices.shape
    v, d = table.shape
    return _make_gather(b, h, v, d)(table, indices)
